# baseline (speedup 1.0000x reference)
import jax
import jax.numpy as jnp
from jax import lax
from jax.experimental import pallas as pl
from jax.experimental.pallas import tpu as pltpu
from jax.experimental.pallas import tpu_sc as plsc

VOCAB = 1000000
HIDDEN = 64
B = 4096
L = 200
N = B * L

NC, NS, LANES = 2, 16, 16
NW = NC * NS
IDX_W = 128
SUB = 4
CHUNK = SUB * IDX_W
PER_W = N // NW
ROWS_PER_W = PER_W // IDX_W
NITER = PER_W // CHUNK
NPAIR = NITER // 2


def _prep_body(dow2_ref, hod2_ref, dow_emb_ref, hod_emb_ref,
               comb_ref, fused_ref):
    comb_ref[...] = dow2_ref[...] * 25 + hod2_ref[...]
    for d in range(8):
        fused_ref[d * 25:(d + 1) * 25, :] = dow_emb_ref[d:d + 1, :] + hod_emb_ref[...]


def _prep(dow2, hod2, dow_emb, hod_emb):
    return pl.pallas_call(
        _prep_body,
        out_shape=[
            jax.ShapeDtypeStruct((N // IDX_W, IDX_W), jnp.int32),
            jax.ShapeDtypeStruct((200, HIDDEN), jnp.float32),
        ],
    )(dow2, hod2, dow_emb, hod_emb)


def _sc_body(enc_hbm, comb_hbm, emb_hbm, fused_hbm, out_hbm,
             enc_v, comb_v, rows0, rows1, sem_g, sem_o0, sem_o1):
    wid = lax.axis_index("s") * NC + lax.axis_index("c")
    row0 = wid * ROWS_PER_W

    pltpu.sync_copy(enc_hbm.at[pl.ds(row0, ROWS_PER_W)], enc_v)
    pltpu.sync_copy(comb_hbm.at[pl.ds(row0, ROWS_PER_W)], comb_v)

    def chunk(t, rows_p, sem_o, drain):
        rbase = t * SUB
        obase = (row0 + t * SUB) * IDX_W

        if drain:
            pltpu.make_async_copy(
                rows_p, out_hbm.at[pl.ds(obase, CHUNK)], sem_o).wait()

        copies = []
        for j in range(SUB):
            dst = pl.ds(j * IDX_W, IDX_W)
            copies.append(pltpu.async_copy(
                emb_hbm.at[enc_v.at[rbase + j]], rows_p.at[dst], sem_g))
        for cp in copies:
            cp.wait()

        copies = []
        for j in range(SUB):
            dst = pl.ds(j * IDX_W, IDX_W)
            copies.append(pltpu.async_copy(
                fused_hbm.at[comb_v.at[rbase + j]], rows_p.at[dst], sem_g,
                add=True))
        for cp in copies:
            cp.wait()

        pltpu.async_copy(rows_p, out_hbm.at[pl.ds(obase, CHUNK)], sem_o)

    def pair(g, drain):
        chunk(2 * g, rows0, sem_o0, drain)
        chunk(2 * g + 1, rows1, sem_o1, drain)

    pair(0, False)
    lax.fori_loop(1, NPAIR, lambda g, c: (pair(g, True), c)[1], 0)

    pltpu.make_async_copy(rows0, out_hbm.at[pl.ds(0, CHUNK)], sem_o0).wait()
    pltpu.make_async_copy(rows1, out_hbm.at[pl.ds(0, CHUNK)], sem_o1).wait()


def _sc_lookup(enc2, comb2, embedding, fused):
    mesh = plsc.VectorSubcoreMesh(core_axis_name="c", subcore_axis_name="s")
    k = pl.kernel(
        _sc_body,
        out_type=jax.ShapeDtypeStruct((N, HIDDEN), jnp.float32),
        mesh=mesh,
        compiler_params=pltpu.CompilerParams(use_tc_tiling_on_sc=False),
        scratch_types=[
            pltpu.VMEM((ROWS_PER_W, IDX_W), jnp.int32),
            pltpu.VMEM((ROWS_PER_W, IDX_W), jnp.int32),
            pltpu.VMEM((CHUNK, HIDDEN), jnp.float32),
            pltpu.VMEM((CHUNK, HIDDEN), jnp.float32),
            pltpu.SemaphoreType.DMA,
            pltpu.SemaphoreType.DMA,
            pltpu.SemaphoreType.DMA,
        ],
    )
    return k(enc2, comb2, embedding, fused)


def kernel(enc_src, DoW, HoD, embedding, DoW_Emb, HoD_Emb):
    enc2 = jnp.asarray(enc_src, jnp.int32).reshape(N // IDX_W, IDX_W)
    dow2 = jnp.asarray(DoW, jnp.int32).reshape(N // IDX_W, IDX_W)
    hod2 = jnp.asarray(HoD, jnp.int32).reshape(N // IDX_W, IDX_W)
    comb2, fused = _prep(dow2, hod2, DoW_Emb.astype(jnp.float32),
                         HoD_Emb.astype(jnp.float32))
    out = _sc_lookup(enc2, comb2, embedding.astype(jnp.float32), fused)
    return out.reshape(B, L, HIDDEN)


# speedup vs baseline: 1.0212x; 1.0212x over previous
import jax
import jax.numpy as jnp
from jax import lax
from jax.experimental import pallas as pl
from jax.experimental.pallas import tpu as pltpu
from jax.experimental.pallas import tpu_sc as plsc

VOCAB = 1000000
HIDDEN = 64
B = 4096
L = 200

NC, NS = 2, 16
NW = NC * NS
BPW = B // NW
RPC = 2
NITER = BPW // RPC
SPLITS = ((0, 128), (128, 72))


def _prep_body(dow_ref, hod_ref, dow_emb_ref, hod_emb_ref,
               comb_ref, fused_ref):
    comb_ref[...] = dow_ref[...] * 25 + hod_ref[...]
    for d in range(8):
        fused_ref[d * 25:(d + 1) * 25, :] = dow_emb_ref[d:d + 1, :] + hod_emb_ref[...]


def _prep(dow, hod, dow_emb, hod_emb):
    return pl.pallas_call(
        _prep_body,
        out_shape=[
            jax.ShapeDtypeStruct((B, L), jnp.int32),
            jax.ShapeDtypeStruct((200, HIDDEN), jnp.float32),
        ],
    )(dow, hod, dow_emb, hod_emb)


def _sc_body(enc_hbm, comb_hbm, emb_hbm, fused_hbm, out_hbm,
             enc0, enc1, comb0, comb1, rows0, rows1,
             sem_i, sem_g, sem_o0, sem_o1):
    wid = lax.axis_index("s") * NC + lax.axis_index("c")
    b0 = wid * BPW

    def chunk(t, encb, combb, rows_p, sem_o, drain):
        bb = b0 + t * RPC

        ci = pltpu.async_copy(enc_hbm.at[pl.ds(bb, RPC)], encb, sem_i)
        cc = pltpu.async_copy(comb_hbm.at[pl.ds(bb, RPC)], combb, sem_i)
        ci.wait()
        cc.wait()

        if drain:
            pltpu.make_async_copy(
                rows_p, out_hbm.at[pl.ds(bb, RPC)], sem_o).wait()

        copies = []
        for i in range(RPC):
            for off, n in SPLITS:
                sl = pl.ds(off, n)
                copies.append(pltpu.async_copy(
                    emb_hbm.at[encb.at[i, sl]], rows_p.at[i, sl], sem_g))
        for cp in copies:
            cp.wait()

        copies = []
        for i in range(RPC):
            for off, n in SPLITS:
                sl = pl.ds(off, n)
                copies.append(pltpu.async_copy(
                    fused_hbm.at[combb.at[i, sl]], rows_p.at[i, sl], sem_g,
                    add=True))
        for cp in copies:
            cp.wait()

        pltpu.async_copy(rows_p, out_hbm.at[pl.ds(bb, RPC)], sem_o)

    def pair(g, drain):
        chunk(2 * g, enc0, comb0, rows0, sem_o0, drain)
        chunk(2 * g + 1, enc1, comb1, rows1, sem_o1, drain)

    pair(0, False)
    lax.fori_loop(1, NITER // 2, lambda g, c: (pair(g, True), c)[1], 0)

    pltpu.make_async_copy(rows0, out_hbm.at[pl.ds(0, RPC)], sem_o0).wait()
    pltpu.make_async_copy(rows1, out_hbm.at[pl.ds(0, RPC)], sem_o1).wait()


def _sc_lookup(enc, comb, embedding, fused):
    mesh = plsc.VectorSubcoreMesh(core_axis_name="c", subcore_axis_name="s")
    k = pl.kernel(
        _sc_body,
        out_type=jax.ShapeDtypeStruct((B, L, HIDDEN), jnp.float32),
        mesh=mesh,
        compiler_params=pltpu.CompilerParams(use_tc_tiling_on_sc=False),
        scratch_types=[
            pltpu.VMEM((RPC, L), jnp.int32),
            pltpu.VMEM((RPC, L), jnp.int32),
            pltpu.VMEM((RPC, L), jnp.int32),
            pltpu.VMEM((RPC, L), jnp.int32),
            pltpu.VMEM((RPC, L, HIDDEN), jnp.float32),
            pltpu.VMEM((RPC, L, HIDDEN), jnp.float32),
            pltpu.SemaphoreType.DMA,
            pltpu.SemaphoreType.DMA,
            pltpu.SemaphoreType.DMA,
            pltpu.SemaphoreType.DMA,
        ],
    )
    return k(enc, comb, embedding, fused)


def kernel(enc_src, DoW, HoD, embedding, DoW_Emb, HoD_Emb):
    enc = jnp.asarray(enc_src, jnp.int32)
    dow = jnp.asarray(DoW, jnp.int32)
    hod = jnp.asarray(HoD, jnp.int32)
    comb, fused = _prep(dow, hod, DoW_Emb.astype(jnp.float32),
                        HoD_Emb.astype(jnp.float32))
    return _sc_lookup(enc, comb, embedding.astype(jnp.float32), fused)


# speedup vs baseline: 1.0223x; 1.0011x over previous
import jax
import jax.numpy as jnp
from jax import lax
from jax.experimental import pallas as pl
from jax.experimental.pallas import tpu as pltpu
from jax.experimental.pallas import tpu_sc as plsc

VOCAB = 1000000
HIDDEN = 64
B = 4096
L = 200

NC, NS = 2, 16
NW = NC * NS
BPW = B // NW
RPC = 2
NITER = BPW // RPC
LP = 256
SPLITS = ((0, 128), (128, 72))


def _prep_body(enc_ref, dow_ref, hod_ref, dow_emb_ref, hod_emb_ref,
               encp_ref, comb_ref, fused_ref):
    encp_ref[...] = jnp.pad(enc_ref[...], ((0, 0), (0, LP - L)))
    comb_ref[...] = jnp.pad(dow_ref[...] * 25 + hod_ref[...],
                            ((0, 0), (0, LP - L)))
    for d in range(8):
        fused_ref[d * 25:(d + 1) * 25, :] = dow_emb_ref[d:d + 1, :] + hod_emb_ref[...]


def _prep(enc, dow, hod, dow_emb, hod_emb):
    return pl.pallas_call(
        _prep_body,
        out_shape=[
            jax.ShapeDtypeStruct((B, LP), jnp.int32),
            jax.ShapeDtypeStruct((B, LP), jnp.int32),
            jax.ShapeDtypeStruct((200, HIDDEN), jnp.float32),
        ],
    )(enc, dow, hod, dow_emb, hod_emb)


def _sc_body(enc_hbm, comb_hbm, emb_hbm, fused_hbm, out_hbm,
             enc0, enc1, comb0, comb1, rows0, rows1,
             sem_i, sem_g, sem_o0, sem_o1):
    wid = lax.axis_index("s") * NC + lax.axis_index("c")
    b0 = wid * BPW

    def chunk(t, encb, combb, rows_p, sem_o, drain):
        bb = b0 + t * RPC

        ci = pltpu.async_copy(enc_hbm.at[pl.ds(bb, RPC)], encb, sem_i)
        cc = pltpu.async_copy(comb_hbm.at[pl.ds(bb, RPC)], combb, sem_i)
        ci.wait()
        cc.wait()

        if drain:
            pltpu.make_async_copy(
                rows_p, out_hbm.at[pl.ds(bb, RPC)], sem_o).wait()

        copies = []
        for i in range(RPC):
            for off, n in SPLITS:
                sl = pl.ds(off, n)
                copies.append(pltpu.async_copy(
                    emb_hbm.at[encb.at[i, sl]], rows_p.at[i, sl], sem_g))
        for cp in copies:
            cp.wait()

        copies = []
        for i in range(RPC):
            for off, n in SPLITS:
                sl = pl.ds(off, n)
                copies.append(pltpu.async_copy(
                    fused_hbm.at[combb.at[i, sl]], rows_p.at[i, sl], sem_g,
                    add=True))
        for cp in copies:
            cp.wait()

        pltpu.async_copy(rows_p, out_hbm.at[pl.ds(bb, RPC)], sem_o)

    def pair(g, drain):
        chunk(2 * g, enc0, comb0, rows0, sem_o0, drain)
        chunk(2 * g + 1, enc1, comb1, rows1, sem_o1, drain)

    pair(0, False)
    lax.fori_loop(1, NITER // 2, lambda g, c: (pair(g, True), c)[1], 0)

    pltpu.make_async_copy(rows0, out_hbm.at[pl.ds(0, RPC)], sem_o0).wait()
    pltpu.make_async_copy(rows1, out_hbm.at[pl.ds(0, RPC)], sem_o1).wait()


def _sc_lookup(enc, comb, embedding, fused):
    mesh = plsc.VectorSubcoreMesh(core_axis_name="c", subcore_axis_name="s")
    k = pl.kernel(
        _sc_body,
        out_type=jax.ShapeDtypeStruct((B, L, HIDDEN), jnp.float32),
        mesh=mesh,
        compiler_params=pltpu.CompilerParams(use_tc_tiling_on_sc=False),
        scratch_types=[
            pltpu.VMEM((RPC, LP), jnp.int32),
            pltpu.VMEM((RPC, LP), jnp.int32),
            pltpu.VMEM((RPC, LP), jnp.int32),
            pltpu.VMEM((RPC, LP), jnp.int32),
            pltpu.VMEM((RPC, L, HIDDEN), jnp.float32),
            pltpu.VMEM((RPC, L, HIDDEN), jnp.float32),
            pltpu.SemaphoreType.DMA,
            pltpu.SemaphoreType.DMA,
            pltpu.SemaphoreType.DMA,
            pltpu.SemaphoreType.DMA,
        ],
    )
    return k(enc, comb, embedding, fused)


def kernel(enc_src, DoW, HoD, embedding, DoW_Emb, HoD_Emb):
    enc = jnp.asarray(enc_src, jnp.int32)
    dow = jnp.asarray(DoW, jnp.int32)
    hod = jnp.asarray(HoD, jnp.int32)
    encp, comb, fused = _prep(enc, dow, hod, DoW_Emb.astype(jnp.float32),
                              HoD_Emb.astype(jnp.float32))
    return _sc_lookup(encp, comb, embedding.astype(jnp.float32), fused)
